# Initial kernel scaffold; baseline (speedup 1.0000x reference)
import jax
import jax.numpy as jnp
from jax.experimental import pallas as pl


def kernel(efeat, nfeat, edge_index, w1, b1, w2, b2, ln_g, ln_b):
    raise NotImplementedError("write your pallas kernel here")


# speedup vs baseline: 3.5990x; 3.5990x over previous
import functools

import jax
import jax.numpy as jnp
from jax.experimental import pallas as pl
from jax.experimental.pallas import tpu as pltpu
from jax.experimental.pallas import tpu_sc as plsc

_GATHER_WINDOW = 128
_EDGE_BLOCK = 2000


def _project_body(nfeat_ref, w1_ref, out_ref):
    out_ref[...] = jnp.dot(nfeat_ref[...], w1_ref[...],
                           preferred_element_type=jnp.float32)


def _project(nfeat, w1):
    n, d = nfeat.shape
    h = w1.shape[1]
    return pl.pallas_call(
        _project_body,
        grid=(2,),
        in_specs=[
            pl.BlockSpec((n, d), lambda j: (0, 0)),
            pl.BlockSpec((d, h), lambda j: (j + 1, 0)),
        ],
        out_specs=pl.BlockSpec((n, h), lambda j: (j, 0)),
        out_shape=jax.ShapeDtypeStruct((2 * n, h), jnp.float32),
    )(nfeat, w1)


def _sc_gather(table, idx):
    num_idx = idx.shape[1]
    h = table.shape[1]
    mesh = plsc.VectorSubcoreMesh(core_axis_name="c", subcore_axis_name="s")

    @functools.partial(
        pl.kernel,
        out_type=jax.ShapeDtypeStruct((num_idx, h), jnp.float32),
        mesh=mesh,
    )
    def gather_kernel(table_hbm, idx_hbm, out_hbm):
        def body(i_vmem, o_vmem):
            pltpu.sync_copy(table_hbm.at[i_vmem.at[0]], o_vmem)

        pltpu.emit_pipeline(
            body,
            grid=(num_idx // _GATHER_WINDOW,),
            in_specs=[pl.BlockSpec((1, _GATHER_WINDOW), lambda i: (0, i))],
            out_specs=[pl.BlockSpec((_GATHER_WINDOW, h), lambda i: (i, 0))],
            core_axis_name=("c", "s"),
            dimension_semantics=(pltpu.PARALLEL,),
        )(idx_hbm, out_hbm)

    return gather_kernel(table, idx)


def _mlp_body(ef_ref, gs_ref, gd_ref, w1_ref, b1_ref, w2_ref, b2_ref,
              lg_ref, lb_ref, out_ref):
    ef = ef_ref[...]
    h = jnp.dot(ef, w1_ref[...], preferred_element_type=jnp.float32)
    h = h + gs_ref[...] + gd_ref[...] + b1_ref[...]
    h = h * jax.lax.logistic(h)
    h = jnp.dot(h, w2_ref[...], preferred_element_type=jnp.float32)
    h = h + b2_ref[...]
    mu = jnp.mean(h, axis=-1, keepdims=True)
    var = jnp.mean((h - mu) * (h - mu), axis=-1, keepdims=True)
    h = (h - mu) * jax.lax.rsqrt(var + 1e-5) * lg_ref[...] + lb_ref[...]
    out_ref[...] = h + ef


def _mlp(efeat, gathered, w1, b1, w2, b2, ln_g, ln_b):
    e, d = efeat.shape
    h = w1.shape[1]
    nblk = e // _EDGE_BLOCK
    return pl.pallas_call(
        _mlp_body,
        grid=(nblk,),
        in_specs=[
            pl.BlockSpec((_EDGE_BLOCK, d), lambda i: (i, 0)),
            pl.BlockSpec((_EDGE_BLOCK, h), lambda i: (i, 0)),
            pl.BlockSpec((_EDGE_BLOCK, h), lambda i: (i + nblk, 0)),
            pl.BlockSpec((d, h), lambda i: (0, 0)),
            pl.BlockSpec((1, h), lambda i: (0, 0)),
            pl.BlockSpec((h, d), lambda i: (0, 0)),
            pl.BlockSpec((1, d), lambda i: (0, 0)),
            pl.BlockSpec((1, d), lambda i: (0, 0)),
            pl.BlockSpec((1, d), lambda i: (0, 0)),
        ],
        out_specs=pl.BlockSpec((_EDGE_BLOCK, d), lambda i: (i, 0)),
        out_shape=jax.ShapeDtypeStruct((e, d), jnp.float32),
    )(efeat, gathered, gathered, w1, b1.reshape(1, h), w2,
      b2.reshape(1, d), ln_g.reshape(1, d), ln_b.reshape(1, d))


def kernel(efeat, nfeat, edge_index, w1, b1, w2, b2, ln_g, ln_b):
    n = nfeat.shape[0]
    e = efeat.shape[0]
    table = _project(nfeat, w1)
    idx = jnp.concatenate([edge_index[0], edge_index[1] + n]).reshape(1, 2 * e)
    gathered = _sc_gather(table, idx)
    efeat_new = _mlp(efeat, gathered, w1, b1, w2, b2, ln_g, ln_b)
    return (efeat_new, nfeat)
